# baseline (speedup 1.0000x reference)
import functools

import jax
import jax.numpy as jnp
from jax import lax
from jax.experimental import pallas as pl
from jax.experimental.pallas import tpu as pltpu
from jax.experimental.pallas import tpu_sc as plsc

_B, _N, _K, _D, _F = 4, 10000, 20, 64, 32
_BN = _B * _N
_NE = _BN * _K
_CHUNKS = 8
_ROWS = _BN // _CHUNKS
_NW = 32
_PB = 16
_EB = _PB * _K
_NBLK = _BN // _PB
_NB_PER_B = _N // _PB
_ITERS = -(-_NBLK // _NW)
_IC = 80
_NIC = _EB // _IC
_G = 4


def _dense_body(x_ref, wef_ref, wnb_ref, pv_ref,
                r_ref, c_ref, t_ref, w_ref, sa_ref):
    x = x_ref[0]
    gscale = pv_ref[0]
    cvec = pv_ref[1]
    g1s = pv_ref[2]
    bb1 = pv_ref[3]
    wsa = pv_ref[4]
    wna = pv_ref[5]
    sa_b = pv_ref[6, 0]
    a0_b = pv_ref[6, 1]
    q = jnp.dot(x, wef_ref[...], preferred_element_type=jnp.float32)
    r = q * gscale[None, :]
    c = r + cvec[None, :]
    nf = jnp.maximum(jnp.dot(x, wnb_ref[...],
                             preferred_element_type=jnp.float32), 0.0)
    nf = nf * g1s[None, :] + bb1[None, :]
    sa = jnp.sum(nf * wsa[None, :], axis=1) + sa_b
    wv = jnp.sum(r * wna[None, :], axis=1)
    t = sa + jnp.sum(c * wna[None, :], axis=1) + a0_b
    r_ref[0] = r
    c_ref[0] = c
    t_ref[0, 0] = t
    w_ref[0, 0] = wv
    sa_ref[0, 0] = sa


def _sc_body(r_hbm, c_hbm, t_hbm, w_hbm, m_hbm, nn_hbm,
             ef_hbm, val_hbm,
             w_tab, m_tab, idxr, idxf, rows, eft, cv, tv, vvt, sem):
    cid = lax.axis_index("c")
    sid = lax.axis_index("s")
    wid = sid * 2 + cid
    iota = lax.iota(jnp.int32, 16)
    ek = [iota * _K + k for k in range(_K)]
    start = wid * 78 + jnp.minimum(wid, 4)
    end = start + jnp.where(wid < 4, 79, 78)

    def grp_body(g, b_prev):
        blk0 = start + g * _G

        def blk_body(j, bp):
            blk = blk0 + j
            valid = blk < end
            blk_c = jnp.minimum(blk, _NBLK - 1)
            b = blk_c // _NB_PER_B

            @pl.when(b != bp)
            def _():
                pltpu.sync_copy(w_hbm.at[pl.ds(b * _N, _N)], w_tab)
                pltpu.sync_copy(m_hbm.at[pl.ds(b * _N, _N)], m_tab)

            @pl.when(valid)
            def _():
                base_p = pl.multiple_of(blk * _PB, 8)
                base_e = pl.multiple_of(blk * _EB, 8)
                pltpu.sync_copy(nn_hbm.at[pl.ds(base_e, _EB)], idxr)
                off = b * _N
                for q in range(_EB // 16):
                    sl = pl.ds(q * 16, 16)
                    idxf[sl] = idxr[sl] + off
                descs = [
                    pltpu.async_copy(
                        r_hbm.at[idxf.at[pl.ds(q * _IC, _IC)]],
                        rows.at[pl.ds(q * _IC, _IC)], sem)
                    for q in range(_NIC)
                ]
                pltpu.sync_copy(c_hbm.at[pl.ds(base_p, _PB)], cv)
                pltpu.sync_copy(t_hbm.at[pl.ds(base_p, _PB)], tv)
                tvec = tv[...]
                ls = []
                for k in range(_K):
                    ik = plsc.load_gather(idxr, [ek[k]])
                    wk = plsc.load_gather(w_tab, [ik])
                    mk = plsc.load_gather(m_tab, [ik])
                    l = tvec - wk
                    l = jnp.where(l > 0, l, 0.2 * l) - 10000.0 * mk
                    ls.append(l)
                m = ls[0]
                for k in range(1, _K):
                    m = jnp.maximum(m, ls[k])
                es = [jnp.exp(l - m) for l in ls]
                ssum = es[0]
                for k in range(1, _K):
                    ssum = ssum + es[k]
                inv = 1.0 / ssum
                cfs = [e * inv for e in es]
                for d in descs:
                    d.wait()

                jcol = j * _PB
                for p in range(_PB):
                    c_lo = cv[p, pl.ds(0, 16)]
                    c_hi = cv[p, pl.ds(16, 16)]
                    fullc = jnp.full((16,), p, jnp.int32) + jcol
                    acc_lo = jnp.zeros((16,), jnp.float32)
                    acc_hi = jnp.zeros((16,), jnp.float32)
                    for k in range(_K):
                        row = p * _K + k
                        e_lo = c_lo - rows[row, pl.ds(0, 16)]
                        e_hi = c_hi - rows[row, pl.ds(16, 16)]
                        plsc.store_scatter(eft, [iota + (k * _F), fullc],
                                           e_lo)
                        plsc.store_scatter(eft,
                                           [iota + (k * _F + 16), fullc],
                                           e_hi)
                        cf = cfs[k][p]
                        acc_lo = acc_lo + cf * e_lo
                        acc_hi = acc_hi + cf * e_hi
                    plsc.store_scatter(vvt, [iota, fullc],
                                       jnp.where(acc_lo > 0, acc_lo,
                                                 0.2 * acc_lo))
                    plsc.store_scatter(vvt, [iota + 16, fullc],
                                       jnp.where(acc_hi > 0, acc_hi,
                                                 0.2 * acc_hi))

            return b

        b_last = lax.fori_loop(0, _G, blk_body, b_prev)

        b0 = blk0 // _NB_PER_B
        n0 = blk0 - b0 * _NB_PER_B
        uniform = jnp.logical_and(blk0 + _G <= end,
                                  n0 + _G <= _NB_PER_B)

        @pl.when(uniform)
        def _():
            base_n = pl.multiple_of(n0 * _PB, 8)
            pltpu.sync_copy(eft,
                            ef_hbm.at[pl.ds(b0 * (_K * _F), _K * _F),
                                      pl.ds(base_n, _G * _PB)])
            pltpu.sync_copy(vvt,
                            val_hbm.at[pl.ds(b0 * _F, _F),
                                       pl.ds(base_n, _G * _PB)])

        @pl.when(jnp.logical_not(uniform))
        def _():
            def fl_body(j, c2):
                blk = blk0 + j

                @pl.when(blk < end)
                def _():
                    bj = blk // _NB_PER_B
                    base_nj = pl.multiple_of(
                        (blk - bj * _NB_PER_B) * _PB, 8)
                    jc = j * _PB
                    pltpu.sync_copy(
                        eft.at[pl.ds(0, _K * _F), pl.ds(jc, _PB)],
                        ef_hbm.at[pl.ds(bj * (_K * _F), _K * _F),
                                  pl.ds(base_nj, _PB)])
                    pltpu.sync_copy(
                        vvt.at[pl.ds(0, _F), pl.ds(jc, _PB)],
                        val_hbm.at[pl.ds(bj * _F, _F),
                                   pl.ds(base_nj, _PB)])

                return c2

            lax.fori_loop(0, _G, fl_body, 0)

        return b_last

    lax.fori_loop(0, 20, grp_body, jnp.int32(-1))


def kernel(inputs, nn_idx, mask, W_nb, g1, bb1, W_ef, b_ef, g2, bb2,
           W_sa, b_sa, g3, bb3, W_na, b_na, g4, bb4):
    f32 = jnp.float32
    s = (1.0 + 1e-3) ** -0.5
    gscale = g2 * s
    pv = jnp.zeros((8, _F), f32)
    pv = pv.at[0].set(gscale)
    pv = pv.at[1].set(gscale * b_ef + bb2)
    pv = pv.at[2].set(g1 * s)
    pv = pv.at[3].set(bb1)
    pv = pv.at[4].set((g3[0] * s) * W_sa[:, 0])
    pv = pv.at[5].set((g4[0] * s) * W_na[:, 0])
    pv = pv.at[6, 0].set(g3[0] * s * b_sa[0] + bb3[0])
    pv = pv.at[6, 1].set(g4[0] * s * b_na[0] + bb4[0])

    x3 = inputs.reshape(_CHUNKS, _ROWS, _D)
    dense = pl.pallas_call(
        _dense_body,
        grid=(_CHUNKS,),
        in_specs=[
            pl.BlockSpec((1, _ROWS, _D), lambda i: (i, 0, 0)),
            pl.BlockSpec((_D, _F), lambda i: (0, 0)),
            pl.BlockSpec((_D, _F), lambda i: (0, 0)),
            pl.BlockSpec((8, _F), lambda i: (0, 0)),
        ],
        out_specs=[
            pl.BlockSpec((1, _ROWS, _F), lambda i: (i, 0, 0)),
            pl.BlockSpec((1, _ROWS, _F), lambda i: (i, 0, 0)),
            pl.BlockSpec((1, 1, _ROWS), lambda i: (i, 0, 0)),
            pl.BlockSpec((1, 1, _ROWS), lambda i: (i, 0, 0)),
            pl.BlockSpec((1, 1, _ROWS), lambda i: (i, 0, 0)),
        ],
        out_shape=[
            jax.ShapeDtypeStruct((_CHUNKS, _ROWS, _F), f32),
            jax.ShapeDtypeStruct((_CHUNKS, _ROWS, _F), f32),
            jax.ShapeDtypeStruct((_CHUNKS, 1, _ROWS), f32),
            jax.ShapeDtypeStruct((_CHUNKS, 1, _ROWS), f32),
            jax.ShapeDtypeStruct((_CHUNKS, 1, _ROWS), f32),
        ],
    )
    r3, c3, t3, w3, sa3 = dense(x3, W_ef, W_nb, pv)

    r_flat = r3.reshape(_BN, _F)
    c_flat = c3.reshape(_BN, _F)
    t_flat = t3.reshape(_BN)
    w_flat = w3.reshape(_BN)
    m_flat = mask.reshape(_BN)
    nn_flat = nn_idx.reshape(_NE).astype(jnp.int32)

    sc = pl.kernel(
        _sc_body,
        out_type=[
            jax.ShapeDtypeStruct((_B * _K * _F, _N), f32),
            jax.ShapeDtypeStruct((_B * _F, _N), f32),
        ],
        mesh=plsc.VectorSubcoreMesh(core_axis_name="c", subcore_axis_name="s",
                                    num_cores=2, num_subcores=16),
        scratch_types=[
            pltpu.VMEM((_N,), f32),
            pltpu.VMEM((_N,), f32),
            pltpu.VMEM((_EB,), jnp.int32),
            pltpu.VMEM((_EB,), jnp.int32),
            pltpu.VMEM((_EB, _F), f32),
            pltpu.VMEM((_K * _F, _G * _PB), f32),
            pltpu.VMEM((_PB, _F), f32),
            pltpu.VMEM((_PB,), f32),
            pltpu.VMEM((_F, _G * _PB), f32),
            pltpu.SemaphoreType.DMA,
        ],
        compiler_params=pltpu.CompilerParams(needs_layout_passes=False,
                                             use_tc_tiling_on_sc=False),
    )
    ef_flat, val_flat = sc(r_flat, c_flat, t_flat, w_flat, m_flat, nn_flat)

    outputs = val_flat.reshape(_B, 1, _F, _N).transpose(0, 3, 1, 2)
    self_attention = sa3.reshape(_B, _N, 1, 1)
    edge_feature = ef_flat.reshape(_B, _K, _F, _N).transpose(0, 3, 1, 2)
    return (outputs, self_attention, edge_feature)


# speedup vs baseline: 1.1517x; 1.1517x over previous
import functools

import jax
import jax.numpy as jnp
from jax import lax
from jax.experimental import pallas as pl
from jax.experimental.pallas import tpu as pltpu
from jax.experimental.pallas import tpu_sc as plsc

_B, _N, _K, _D, _F = 4, 10000, 20, 64, 32
_BN = _B * _N
_NE = _BN * _K
_CHUNKS = 8
_ROWS = _BN // _CHUNKS
_NW = 32
_PB = 16
_EB = _PB * _K
_NBLK = _BN // _PB
_NB_PER_B = _N // _PB
_ITERS = -(-_NBLK // _NW)
_IC = 80
_NIC = _EB // _IC


def _dense_body(x_ref, wef_ref, wnb_ref, pv_ref,
                r_ref, c_ref, t_ref, w_ref, sa_ref):
    x = x_ref[0]
    gscale = pv_ref[0]
    cvec = pv_ref[1]
    g1s = pv_ref[2]
    bb1 = pv_ref[3]
    wsa = pv_ref[4]
    wna = pv_ref[5]
    sa_b = pv_ref[6, 0]
    a0_b = pv_ref[6, 1]
    q = jnp.dot(x, wef_ref[...], preferred_element_type=jnp.float32)
    r = q * gscale[None, :]
    c = r + cvec[None, :]
    nf = jnp.maximum(jnp.dot(x, wnb_ref[...],
                             preferred_element_type=jnp.float32), 0.0)
    nf = nf * g1s[None, :] + bb1[None, :]
    sa = jnp.sum(nf * wsa[None, :], axis=1) + sa_b
    wv = jnp.sum(r * wna[None, :], axis=1)
    t = sa + jnp.sum(c * wna[None, :], axis=1) + a0_b
    r_ref[0] = r
    c_ref[0] = c
    t_ref[0, 0] = t
    w_ref[0, 0] = wv
    sa_ref[0, 0] = sa


def _sc_body(r_hbm, c_hbm, t_hbm, w_hbm, m_hbm, nn_hbm,
             ef_hbm, val_hbm,
             w_tab, m_tab, idxf, rows, efv, cv, tv, vv, sem):
    cid = lax.axis_index("c")
    sid = lax.axis_index("s")
    wid = sid * 2 + cid
    pltpu.sync_copy(w_hbm, w_tab)
    pltpu.sync_copy(m_hbm, m_tab)
    iota = lax.iota(jnp.int32, 16)
    ek = [iota * _K + k for k in range(_K)]

    def blk_body(i, carry):
        blk = wid + i * _NW

        @pl.when(blk < _NBLK)
        def _():
            b = blk // _NB_PER_B
            base_p = pl.multiple_of(blk * _PB, 8)
            base_e = pl.multiple_of(blk * _EB, 8)
            base_p4 = pl.multiple_of(blk * (_PB // 4), 4)
            base_e4 = pl.multiple_of(blk * (_EB // 4), 8)
            pltpu.sync_copy(nn_hbm.at[pl.ds(base_e, _EB)], idxf)
            off = b * _N
            for j in range(_EB // 16):
                sl = pl.ds(j * 16, 16)
                idxf[sl] = idxf[sl] + off
            descs = [
                pltpu.async_copy(
                    r_hbm.at[idxf.at[pl.ds(j * _IC, _IC)]],
                    rows.at[pl.ds(j * _IC, _IC)], sem)
                for j in range(_NIC)
            ]
            pltpu.sync_copy(c_hbm.at[pl.ds(base_p, _PB)], cv)
            pltpu.sync_copy(t_hbm.at[pl.ds(base_p, _PB)], tv)
            tvec = tv[...]
            ls = []
            for k in range(_K):
                ik = plsc.load_gather(idxf, [ek[k]])
                wk = plsc.load_gather(w_tab, [ik])
                mk = plsc.load_gather(m_tab, [ik])
                l = tvec - wk
                l = jnp.where(l > 0, l, 0.2 * l) - 10000.0 * mk
                ls.append(l)
            m = ls[0]
            for k in range(1, _K):
                m = jnp.maximum(m, ls[k])
            es = [jnp.exp(l - m) for l in ls]
            ssum = es[0]
            for k in range(1, _K):
                ssum = ssum + es[k]
            inv = 1.0 / ssum
            cfs = [e * inv for e in es]
            for d in descs:
                d.wait()

            for p in range(_PB):
                c_lo = cv[p, pl.ds(0, 16)]
                c_hi = cv[p, pl.ds(16, 16)]
                acc_lo = jnp.zeros((16,), jnp.float32)
                acc_hi = jnp.zeros((16,), jnp.float32)
                for k in range(_K):
                    row = p * _K + k
                    e_lo = c_lo - rows[row, pl.ds(0, 16)]
                    e_hi = c_hi - rows[row, pl.ds(16, 16)]
                    efv[row // 4, pl.ds((row % 4) * 32, 16)] = e_lo
                    efv[row // 4, pl.ds((row % 4) * 32 + 16, 16)] = e_hi
                    cf = cfs[k][p]
                    acc_lo = acc_lo + cf * e_lo
                    acc_hi = acc_hi + cf * e_hi
                vv[p // 4, pl.ds((p % 4) * 32, 16)] = jnp.where(
                    acc_lo > 0, acc_lo, 0.2 * acc_lo)
                vv[p // 4, pl.ds((p % 4) * 32 + 16, 16)] = jnp.where(
                    acc_hi > 0, acc_hi, 0.2 * acc_hi)
            pltpu.sync_copy(efv, ef_hbm.at[pl.ds(base_e4, _EB // 4)])
            pltpu.sync_copy(vv, val_hbm.at[pl.ds(base_p4, _PB // 4)])

        return carry

    lax.fori_loop(0, _ITERS, blk_body, 0)


def kernel(inputs, nn_idx, mask, W_nb, g1, bb1, W_ef, b_ef, g2, bb2,
           W_sa, b_sa, g3, bb3, W_na, b_na, g4, bb4):
    f32 = jnp.float32
    s = (1.0 + 1e-3) ** -0.5
    gscale = g2 * s
    pv = jnp.zeros((8, _F), f32)
    pv = pv.at[0].set(gscale)
    pv = pv.at[1].set(gscale * b_ef + bb2)
    pv = pv.at[2].set(g1 * s)
    pv = pv.at[3].set(bb1)
    pv = pv.at[4].set((g3[0] * s) * W_sa[:, 0])
    pv = pv.at[5].set((g4[0] * s) * W_na[:, 0])
    pv = pv.at[6, 0].set(g3[0] * s * b_sa[0] + bb3[0])
    pv = pv.at[6, 1].set(g4[0] * s * b_na[0] + bb4[0])

    x3 = inputs.reshape(_CHUNKS, _ROWS, _D)
    dense = pl.pallas_call(
        _dense_body,
        grid=(_CHUNKS,),
        in_specs=[
            pl.BlockSpec((1, _ROWS, _D), lambda i: (i, 0, 0)),
            pl.BlockSpec((_D, _F), lambda i: (0, 0)),
            pl.BlockSpec((_D, _F), lambda i: (0, 0)),
            pl.BlockSpec((8, _F), lambda i: (0, 0)),
        ],
        out_specs=[
            pl.BlockSpec((1, _ROWS, _F), lambda i: (i, 0, 0)),
            pl.BlockSpec((1, _ROWS, _F), lambda i: (i, 0, 0)),
            pl.BlockSpec((1, 1, _ROWS), lambda i: (i, 0, 0)),
            pl.BlockSpec((1, 1, _ROWS), lambda i: (i, 0, 0)),
            pl.BlockSpec((1, 1, _ROWS), lambda i: (i, 0, 0)),
        ],
        out_shape=[
            jax.ShapeDtypeStruct((_CHUNKS, _ROWS, _F), f32),
            jax.ShapeDtypeStruct((_CHUNKS, _ROWS, _F), f32),
            jax.ShapeDtypeStruct((_CHUNKS, 1, _ROWS), f32),
            jax.ShapeDtypeStruct((_CHUNKS, 1, _ROWS), f32),
            jax.ShapeDtypeStruct((_CHUNKS, 1, _ROWS), f32),
        ],
    )
    r3, c3, t3, w3, sa3 = dense(x3, W_ef, W_nb, pv)

    r_flat = r3.reshape(_BN, _F)
    c_flat = c3.reshape(_BN, _F)
    t_flat = t3.reshape(_BN)
    w_flat = w3.reshape(_BN)
    m_flat = mask.reshape(_BN)
    nn_flat = nn_idx.reshape(_NE).astype(jnp.int32)

    sc = pl.kernel(
        _sc_body,
        out_type=[
            jax.ShapeDtypeStruct((_NE // 4, 128), f32),
            jax.ShapeDtypeStruct((_BN // 4, 128), f32),
        ],
        mesh=plsc.VectorSubcoreMesh(core_axis_name="c", subcore_axis_name="s",
                                    num_cores=2, num_subcores=16),
        scratch_types=[
            pltpu.VMEM((_BN,), f32),
            pltpu.VMEM((_BN,), f32),
            pltpu.VMEM((_EB,), jnp.int32),
            pltpu.VMEM((_EB, _F), f32),
            pltpu.VMEM((_EB // 4, 128), f32),
            pltpu.VMEM((_PB, _F), f32),
            pltpu.VMEM((_PB,), f32),
            pltpu.VMEM((_PB // 4, 128), f32),
            pltpu.SemaphoreType.DMA,
        ],
        compiler_params=pltpu.CompilerParams(needs_layout_passes=False,
                                             use_tc_tiling_on_sc=False),
    )
    ef_flat, val_flat = sc(r_flat, c_flat, t_flat, w_flat, m_flat, nn_flat)

    outputs = val_flat.reshape(_B, _N, 1, _F)
    self_attention = sa3.reshape(_B, _N, 1, 1)
    edge_feature = ef_flat.reshape(_B, _N, _K, _F)
    return (outputs, self_attention, edge_feature)
